# baseline (speedup 1.0000x reference)
import functools

import jax
import jax.numpy as jnp
from jax import lax
from jax.experimental import pallas as pl
from jax.experimental.pallas import tpu as pltpu
from jax.experimental.pallas import tpu_sc as plsc

_BN = 512


def _argmin_body(x_ref, e2_ref, o_ref, nee_ref):
    k = e2_ref.shape[0]

    @pl.when(pl.program_id(0) == 0)
    def _():
        e2 = e2_ref[...]
        nee_ref[...] = -0.25 * jnp.sum(e2 * e2, axis=1, keepdims=True)

    x = x_ref[...]
    dot2t = lax.dot_general(e2_ref[...], x, (((1,), (1,)), ((), ())),
                            preferred_element_type=jnp.float32)
    xx = jnp.sum(x * x, axis=1)
    dist = (dot2t - xx[None, :]) + nee_ref[...]
    o_ref[0, 0, :] = jnp.argmax(dist, axis=0).astype(jnp.int32)


def _argmin_indices(xf, e2, blk0, nb):
    n, d = xf.shape
    k = e2.shape[0]
    out = pl.pallas_call(
        _argmin_body,
        grid=(nb,),
        in_specs=[
            pl.BlockSpec((_BN, d), lambda i: (i + blk0, 0)),
            pl.BlockSpec((k, d), lambda i: (0, 0)),
        ],
        out_specs=pl.BlockSpec((1, 1, _BN), lambda i: (i, 0, 0)),
        out_shape=jax.ShapeDtypeStruct((nb, 1, _BN), jnp.int32),
        scratch_shapes=[pltpu.VMEM((k, 1), jnp.float32)],
    )(xf, e2)
    return out.reshape(nb * _BN)


def _gather_rows(table, idx):
    info = plsc.get_sparse_core_info()
    nc, ns = info.num_cores, info.num_subcores
    nw = nc * ns
    n = idx.shape[0]
    d = table.shape[1]
    b_per_w = n // nw
    ch = 144
    nch = b_per_w // ch
    nbuf = min(3, nch)
    mesh = plsc.VectorSubcoreMesh(core_axis_name="c", subcore_axis_name="s")

    @functools.partial(
        pl.kernel, mesh=mesh,
        out_type=jax.ShapeDtypeStruct((n, d), jnp.float32),
        scratch_types=[
            pltpu.VMEM((b_per_w,), jnp.int32),
            pltpu.VMEM((nbuf, ch, d), jnp.float32),
            pltpu.SemaphoreType.DMA((nbuf,)),
            pltpu.SemaphoreType.DMA((nbuf,)),
        ],
    )
    def k(table_hbm, idx_hbm, out_hbm, idx_v, rows_v, gsem, ssem):
        wid = lax.axis_index("s") * nc + lax.axis_index("c")
        base = wid * b_per_w
        pltpu.sync_copy(idx_hbm.at[pl.ds(base, b_per_w)], idx_v)
        gathers = [None] * nch
        stores = [None] * nch
        for c in range(min(nbuf, nch)):
            gathers[c] = pltpu.async_copy(
                table_hbm.at[idx_v.at[pl.ds(c * ch, ch)]], rows_v.at[c], gsem.at[c])
        for c in range(nch):
            b = c % nbuf
            gathers[c].wait()
            stores[c] = pltpu.async_copy(
                rows_v.at[b], out_hbm.at[pl.ds(base + c * ch, ch)], ssem.at[b])
            nxt = c + nbuf
            if nxt < nch:
                stores[c].wait()
                gathers[nxt] = pltpu.async_copy(
                    table_hbm.at[idx_v.at[pl.ds(nxt * ch, ch)]], rows_v.at[b],
                    gsem.at[b])
        for c in range(max(0, nch - nbuf), nch):
            if stores[c] is not None:
                stores[c].wait()

    return k(table, idx)


def kernel(x, embed):
    shape = x.shape
    d = shape[-1]
    xf = x.reshape(-1, d)
    e2 = embed + embed
    n = xf.shape[0]
    nblk = n // _BN
    nchunk = 4
    blk_per_chunk = nblk // nchunk
    qs, idxs = [], []
    for t in range(nchunk):
        idx_c = _argmin_indices(xf, e2, t * blk_per_chunk, blk_per_chunk)
        qs.append(_gather_rows(embed, idx_c))
        idxs.append(idx_c)
    quantize = jnp.concatenate(qs, axis=0)
    idx = jnp.concatenate(idxs, axis=0)
    return (quantize.reshape(shape), idx.reshape(shape[:-1]))


# speedup vs baseline: 1.5177x; 1.5177x over previous
import functools

import jax
import jax.numpy as jnp
from jax import lax
from jax.experimental import pallas as pl
from jax.experimental.pallas import tpu as pltpu
from jax.experimental.pallas import tpu_sc as plsc

_BN = 1024


def _argmin_body(x_ref, e2_ref, o_ref, nee_ref):
    k = e2_ref.shape[0]

    @pl.when(pl.program_id(0) == 0)
    def _():
        e2 = e2_ref[...]
        nee_ref[...] = -0.25 * jnp.sum(e2 * e2, axis=1, keepdims=True)

    x = x_ref[...]
    dot2t = lax.dot_general(e2_ref[...], x, (((1,), (1,)), ((), ())),
                            preferred_element_type=jnp.float32)
    xx = jnp.sum(x * x, axis=1)
    dist = (dot2t - xx[None, :]) + nee_ref[...]
    o_ref[0, 0, :] = jnp.argmax(dist, axis=0).astype(jnp.int32)


def _argmin_indices(xf, e2, blk0, nb):
    n, d = xf.shape
    k = e2.shape[0]
    out = pl.pallas_call(
        _argmin_body,
        grid=(nb,),
        in_specs=[
            pl.BlockSpec((_BN, d), lambda i: (i + blk0, 0)),
            pl.BlockSpec((k, d), lambda i: (0, 0)),
        ],
        out_specs=pl.BlockSpec((1, 1, _BN), lambda i: (i, 0, 0)),
        out_shape=jax.ShapeDtypeStruct((nb, 1, _BN), jnp.int32),
        scratch_shapes=[pltpu.VMEM((k, 1), jnp.float32)],
    )(xf, e2)
    return out.reshape(nb * _BN)


def _gather_rows(table, idx):
    info = plsc.get_sparse_core_info()
    nc, ns = info.num_cores, info.num_subcores
    nw = nc * ns
    n = idx.shape[0]
    d = table.shape[1]
    b_per_w = n // nw
    ch = 144
    nch = b_per_w // ch
    nbuf = min(3, nch)
    mesh = plsc.VectorSubcoreMesh(core_axis_name="c", subcore_axis_name="s")

    @functools.partial(
        pl.kernel, mesh=mesh,
        out_type=jax.ShapeDtypeStruct((n, d), jnp.float32),
        scratch_types=[
            pltpu.VMEM((b_per_w,), jnp.int32),
            pltpu.VMEM((nbuf, ch, d), jnp.float32),
            pltpu.SemaphoreType.DMA((nbuf,)),
            pltpu.SemaphoreType.DMA((nbuf,)),
        ],
    )
    def k(table_hbm, idx_hbm, out_hbm, idx_v, rows_v, gsem, ssem):
        wid = lax.axis_index("s") * nc + lax.axis_index("c")
        base = wid * b_per_w
        pltpu.sync_copy(idx_hbm.at[pl.ds(base, b_per_w)], idx_v)
        gathers = [None] * nch
        stores = [None] * nch
        for c in range(min(nbuf, nch)):
            gathers[c] = pltpu.async_copy(
                table_hbm.at[idx_v.at[pl.ds(c * ch, ch)]], rows_v.at[c], gsem.at[c])
        for c in range(nch):
            b = c % nbuf
            gathers[c].wait()
            stores[c] = pltpu.async_copy(
                rows_v.at[b], out_hbm.at[pl.ds(base + c * ch, ch)], ssem.at[b])
            nxt = c + nbuf
            if nxt < nch:
                stores[c].wait()
                gathers[nxt] = pltpu.async_copy(
                    table_hbm.at[idx_v.at[pl.ds(nxt * ch, ch)]], rows_v.at[b],
                    gsem.at[b])
        for c in range(max(0, nch - nbuf), nch):
            if stores[c] is not None:
                stores[c].wait()

    return k(table, idx)


def kernel(x, embed):
    shape = x.shape
    d = shape[-1]
    xf = x.reshape(-1, d)
    e2 = embed + embed
    n = xf.shape[0]
    idx = _argmin_indices(xf, e2, 0, n // _BN)
    quantize = _gather_rows(embed, idx)
    return (quantize.reshape(shape), idx.reshape(shape[:-1]))
